# baseline (speedup 1.0000x reference)
import functools

import jax
import jax.numpy as jnp
from jax import lax
from jax.experimental import pallas as pl
from jax.experimental.pallas import tpu as pltpu
from jax.experimental.pallas import tpu_sc as plsc

_BN = 256
_BK = 2048


def _stats_body(scale_ref, z_ref, emb_ref, idx_ref, ppl_ref,
                en_full, zn_s, ebuf, sacc, mval, midx, avg_acc, sem,
                *, n_total, k_total, bn, bk, nb_count, kb_count):
    nb = pl.program_id(0)
    kb = pl.program_id(1)

    @pl.when((nb == 0) & (kb == 0))
    def _init_once():
        pltpu.make_async_copy(emb_ref, en_full, sem).start()
        avg_acc[...] = jnp.zeros_like(avg_acc)
        pltpu.make_async_copy(emb_ref, en_full, sem).wait()

    @pl.when(nb == 0)
    def _norm_emb_block():
        e = en_full[pl.ds(kb * bk, bk), :]
        en_full[pl.ds(kb * bk, bk), :] = e / jnp.maximum(
            jnp.sqrt(jnp.sum(e * e, axis=1, keepdims=True)), 1e-12)

    scale = scale_ref[0, 0]

    @pl.when(kb == 0)
    def _init_row():
        z = z_ref[...]
        zn_s[...] = z / jnp.maximum(
            jnp.sqrt(jnp.sum(z * z, axis=1, keepdims=True)), 1e-12)
        sacc[...] = jnp.zeros_like(sacc)
        mval[...] = jnp.full_like(mval, -jnp.inf)
        midx[...] = jnp.zeros_like(midx)

    zn = zn_s[...]
    en = en_full[pl.ds(kb * bk, bk), :]
    logits = scale * lax.dot_general(
        zn, en, (((1,), (1,)), ((), ())),
        preferred_element_type=jnp.float32)
    absb = jnp.abs(scale)
    base = kb * bk
    hh = 64
    nj = bk // 128
    lane = lax.broadcasted_iota(jnp.int32, (hh, 128), 1)
    tm_parts, cand_parts = [], []
    for hi in range(bn // hh):
        lg = logits[hi * hh:(hi + 1) * hh, :]
        run_max = lg[:, 0:128]
        ebuf[hi * hh:(hi + 1) * hh, pl.ds(base, 128)] = jnp.exp(
            run_max - absb).astype(jnp.bfloat16)
        run_col = jnp.zeros((hh, 128), jnp.int32)
        for j in range(1, nj):
            v = lg[:, j * 128:(j + 1) * 128]
            ebuf[hi * hh:(hi + 1) * hh, pl.ds(base + j * 128, 128)] = (
                jnp.exp(v - absb).astype(jnp.bfloat16))
            m = v > run_max
            run_max = jnp.where(m, v, run_max)
            run_col = jnp.where(m, j, run_col)
        maxv = jnp.max(run_max, axis=1, keepdims=True)
        gidx = run_col * 128 + lane
        cand_parts.append(jnp.min(
            jnp.where(run_max == maxv, gidx, k_total),
            axis=1, keepdims=True))
        tm_parts.append(maxv)
    tmax = jnp.concatenate(tm_parts, axis=0)
    cand = jnp.concatenate(cand_parts, axis=0) + base
    ones_col = jnp.ones((bk, 1), jnp.bfloat16)
    sacc[...] += lax.dot_general(
        ebuf[:, pl.ds(base, bk)], ones_col, (((1,), (0,)), ((), ())),
        preferred_element_type=jnp.float32)

    better = tmax > mval[...]
    midx[...] = jnp.where(better, cand, midx[...])
    mval[...] = jnp.where(better, tmax, mval[...])

    @pl.when(kb == kb_count - 1)
    def _finish_row():
        recip_row = jnp.transpose(1.0 / sacc[...]).astype(jnp.bfloat16)
        for j in range(kb_count):
            chunk = ebuf[:, pl.ds(j * bk, bk)]
            avg_acc[:, pl.ds(j * bk, bk)] += lax.dot_general(
                recip_row, chunk, (((1,), (0,)), ((), ())),
                preferred_element_type=jnp.float32)
        idx_ref[...] = midx[...]

        @pl.when(nb == nb_count - 1)
        def _finish_all():
            avg = avg_acc[...] / n_total
            ent = jnp.sum(avg * jnp.log(avg + 1e-10))
            ppl_ref[...] = jnp.exp(-ent).reshape(1, 1)


def _stats_call(z_e, embeddings, scale2d):
    n, d = z_e.shape
    k = embeddings.shape[0]
    nb_count = n // _BN
    kb_count = k // _BK
    body = functools.partial(
        _stats_body, n_total=n, k_total=k, bn=_BN, bk=_BK,
        nb_count=nb_count, kb_count=kb_count)
    return pl.pallas_call(
        body,
        grid=(nb_count, kb_count),
        in_specs=[
            pl.BlockSpec((1, 1), lambda i, j: (0, 0)),
            pl.BlockSpec((_BN, d), lambda i, j: (i, 0)),
            pl.BlockSpec(memory_space=pl.ANY),
        ],
        out_specs=[
            pl.BlockSpec((_BN, 1), lambda i, j: (i, 0)),
            pl.BlockSpec((1, 1), lambda i, j: (0, 0)),
        ],
        out_shape=[
            jax.ShapeDtypeStruct((n, 1), jnp.int32),
            jax.ShapeDtypeStruct((1, 1), jnp.float32),
        ],
        scratch_shapes=[
            pltpu.VMEM((k, d), jnp.float32),
            pltpu.VMEM((_BN, d), jnp.float32),
            pltpu.VMEM((_BN, k), jnp.bfloat16),
            pltpu.VMEM((_BN, 1), jnp.float32),
            pltpu.VMEM((_BN, 1), jnp.float32),
            pltpu.VMEM((_BN, 1), jnp.int32),
            pltpu.VMEM((1, k), jnp.float32),
            pltpu.SemaphoreType.DMA,
        ],
        compiler_params=pltpu.CompilerParams(
            dimension_semantics=("arbitrary", "arbitrary")),
    )(scale2d, z_e, embeddings)


def _gather_call(table, idx2d, n, d):
    info = plsc.get_sparse_core_info()
    nc, ns = info.num_cores, info.num_subcores
    nw = nc * ns
    b_per_w = n // nw
    chunks = b_per_w // 128
    mesh = plsc.VectorSubcoreMesh(core_axis_name="c", subcore_axis_name="s")

    @functools.partial(
        pl.kernel, mesh=mesh,
        out_type=jax.ShapeDtypeStruct((n, d), jnp.float32),
        compiler_params=pltpu.CompilerParams(use_tc_tiling_on_sc=False),
        scratch_types=[
            pltpu.VMEM((chunks, 128), jnp.int32),
            pltpu.VMEM((b_per_w, d), jnp.float32),
            pltpu.SemaphoreType.DMA,
        ],
    )
    def _gather_kernel(table_hbm, idx_hbm, out_hbm, idx_v, rows_v, sem):
        wid = lax.axis_index("s") * nc + lax.axis_index("c")
        pltpu.sync_copy(idx_hbm.at[pl.ds(wid * chunks, chunks)], idx_v)
        copies = [
            pltpu.async_copy(table_hbm.at[idx_v.at[j]],
                             rows_v.at[pl.ds(j * 128, 128)], sem)
            for j in range(chunks)
        ]
        for c in copies:
            c.wait()
        pltpu.sync_copy(rows_v, out_hbm.at[pl.ds(wid * b_per_w, b_per_w)])

    return _gather_kernel(table, idx2d)


def _blend_body(rw_ref, z_ref, g_ref, out_ref, alpha_ref):
    a = 1.0 / (1.0 + jnp.exp(-rw_ref[0, 0]))
    out_ref[...] = a * g_ref[...] + (1.0 - a) * z_ref[...]

    @pl.when(pl.program_id(0) == 0)
    def _():
        alpha_ref[...] = a.reshape(1, 1)


def _blend_call(rw2d, z_e, zq_pure):
    n, d = z_e.shape
    bn = 2048
    return pl.pallas_call(
        _blend_body,
        grid=(n // bn,),
        in_specs=[
            pl.BlockSpec((1, 1), lambda i: (0, 0)),
            pl.BlockSpec((bn, d), lambda i: (i, 0)),
            pl.BlockSpec((bn, d), lambda i: (i, 0)),
        ],
        out_specs=[
            pl.BlockSpec((bn, d), lambda i: (i, 0)),
            pl.BlockSpec((1, 1), lambda i: (0, 0)),
        ],
        out_shape=[
            jax.ShapeDtypeStruct((n, d), jnp.float32),
            jax.ShapeDtypeStruct((1, 1), jnp.float32),
        ],
        compiler_params=pltpu.CompilerParams(
            dimension_semantics=("arbitrary",)),
    )(rw2d, z_e, zq_pure)


def kernel(z_e, embeddings, logit_scale, residual_weight):
    n, d = z_e.shape
    scale2d = jnp.reshape(logit_scale, (1, 1)).astype(jnp.float32)
    rw2d = jnp.reshape(residual_weight, (1, 1)).astype(jnp.float32)

    idx_col, ppl = _stats_call(z_e, embeddings, scale2d)
    indices = jnp.reshape(idx_col, (n,))

    zq_pure = _gather_call(embeddings, jnp.reshape(indices, (-1, 128)), n, d)
    z_q, alpha2d = _blend_call(rw2d, z_e, zq_pure)

    perplexity = jnp.reshape(ppl, ())
    alpha = jnp.reshape(alpha2d, ())
    commitment_loss = jnp.zeros((), jnp.float32)
    return (z_q, indices, perplexity, alpha, commitment_loss)


# speedup vs baseline: 1.0148x; 1.0148x over previous
import functools

import jax
import jax.numpy as jnp
from jax import lax
from jax.experimental import pallas as pl
from jax.experimental.pallas import tpu as pltpu
from jax.experimental.pallas import tpu_sc as plsc

_BN = 256
_BK = 2048


def _stats_body(scale_ref, z_ref, emb_ref, idx_ref, ppl_ref,
                en_full, zn_s, ebuf, sacc, mval, midx, avg_acc, sem,
                *, n_total, k_total, bn, bk, nb_count, kb_count):
    nb = pl.program_id(0)
    kb = pl.program_id(1)

    @pl.when((nb == 0) & (kb == 0))
    def _init_once():
        pltpu.make_async_copy(emb_ref, en_full, sem).start()
        avg_acc[...] = jnp.zeros_like(avg_acc)
        pltpu.make_async_copy(emb_ref, en_full, sem).wait()

    @pl.when(nb == 0)
    def _norm_emb_block():
        e = en_full[pl.ds(kb * bk, bk), :]
        en_full[pl.ds(kb * bk, bk), :] = e / jnp.maximum(
            jnp.sqrt(jnp.sum(e * e, axis=1, keepdims=True)), 1e-12)

    scale = scale_ref[0, 0]

    @pl.when(kb == 0)
    def _init_row():
        z = z_ref[...]
        zn_s[...] = z / jnp.maximum(
            jnp.sqrt(jnp.sum(z * z, axis=1, keepdims=True)), 1e-12)
        sacc[...] = jnp.zeros_like(sacc)
        mval[...] = jnp.full_like(mval, -jnp.inf)
        midx[...] = jnp.zeros_like(midx)

    zn = zn_s[...]
    absb = jnp.abs(scale)
    sub = bk // 2
    ones_col = jnp.ones((sub, 1), jnp.bfloat16)
    for half in range(2):
        base = kb * bk + half * sub
        en = en_full[pl.ds(base, sub), :]
        logits = scale * lax.dot_general(
            zn, en, (((1,), (1,)), ((), ())),
            preferred_element_type=jnp.float32)
        ex = jnp.exp(logits - absb).astype(jnp.bfloat16)
        ebuf[:, pl.ds(base, sub)] = ex
        sacc[...] += lax.dot_general(
            ex, ones_col, (((1,), (0,)), ((), ())),
            preferred_element_type=jnp.float32)
        tmax = jnp.max(logits, axis=1, keepdims=True)
        colidx = lax.broadcasted_iota(jnp.int32, (bn, sub), 1)
        cand = jnp.min(jnp.where(logits == tmax, colidx, k_total),
                       axis=1, keepdims=True) + base
        better = tmax > mval[...]
        midx[...] = jnp.where(better, cand, midx[...])
        mval[...] = jnp.where(better, tmax, mval[...])

    @pl.when(kb == kb_count - 1)
    def _finish_row():
        recip_row = jnp.transpose(1.0 / sacc[...]).astype(jnp.bfloat16)
        for j in range(kb_count):
            chunk = ebuf[:, pl.ds(j * bk, bk)]
            avg_acc[:, pl.ds(j * bk, bk)] += lax.dot_general(
                recip_row, chunk, (((1,), (0,)), ((), ())),
                preferred_element_type=jnp.float32)
        idx_ref[...] = midx[...]

        @pl.when(nb == nb_count - 1)
        def _finish_all():
            avg = avg_acc[...] / n_total
            ent = jnp.sum(avg * jnp.log(avg + 1e-10))
            ppl_ref[...] = jnp.exp(-ent).reshape(1, 1)


def _stats_call(z_e, embeddings, scale2d):
    n, d = z_e.shape
    k = embeddings.shape[0]
    nb_count = n // _BN
    kb_count = k // _BK
    body = functools.partial(
        _stats_body, n_total=n, k_total=k, bn=_BN, bk=_BK,
        nb_count=nb_count, kb_count=kb_count)
    return pl.pallas_call(
        body,
        grid=(nb_count, kb_count),
        in_specs=[
            pl.BlockSpec((1, 1), lambda i, j: (0, 0)),
            pl.BlockSpec((_BN, d), lambda i, j: (i, 0)),
            pl.BlockSpec(memory_space=pl.ANY),
        ],
        out_specs=[
            pl.BlockSpec((_BN, 1), lambda i, j: (i, 0)),
            pl.BlockSpec((1, 1), lambda i, j: (0, 0)),
        ],
        out_shape=[
            jax.ShapeDtypeStruct((n, 1), jnp.int32),
            jax.ShapeDtypeStruct((1, 1), jnp.float32),
        ],
        scratch_shapes=[
            pltpu.VMEM((k, d), jnp.float32),
            pltpu.VMEM((_BN, d), jnp.float32),
            pltpu.VMEM((_BN, k), jnp.bfloat16),
            pltpu.VMEM((_BN, 1), jnp.float32),
            pltpu.VMEM((_BN, 1), jnp.float32),
            pltpu.VMEM((_BN, 1), jnp.int32),
            pltpu.VMEM((1, k), jnp.float32),
            pltpu.SemaphoreType.DMA,
        ],
        compiler_params=pltpu.CompilerParams(
            dimension_semantics=("arbitrary", "arbitrary")),
    )(scale2d, z_e, embeddings)


def _gather_call(table, idx2d, n, d):
    info = plsc.get_sparse_core_info()
    nc, ns = info.num_cores, info.num_subcores
    nw = nc * ns
    b_per_w = n // nw
    chunks = b_per_w // 128
    mesh = plsc.VectorSubcoreMesh(core_axis_name="c", subcore_axis_name="s")

    @functools.partial(
        pl.kernel, mesh=mesh,
        out_type=jax.ShapeDtypeStruct((n, d), jnp.float32),
        compiler_params=pltpu.CompilerParams(use_tc_tiling_on_sc=False),
        scratch_types=[
            pltpu.VMEM((chunks, 128), jnp.int32),
            pltpu.VMEM((b_per_w, d), jnp.float32),
            pltpu.SemaphoreType.DMA,
        ],
    )
    def _gather_kernel(table_hbm, idx_hbm, out_hbm, idx_v, rows_v, sem):
        wid = lax.axis_index("s") * nc + lax.axis_index("c")
        pltpu.sync_copy(idx_hbm.at[pl.ds(wid * chunks, chunks)], idx_v)
        copies = [
            pltpu.async_copy(table_hbm.at[idx_v.at[j]],
                             rows_v.at[pl.ds(j * 128, 128)], sem)
            for j in range(chunks)
        ]
        for c in copies:
            c.wait()
        pltpu.sync_copy(rows_v, out_hbm.at[pl.ds(wid * b_per_w, b_per_w)])

    return _gather_kernel(table, idx2d)


def _blend_body(rw_ref, z_ref, g_ref, out_ref, alpha_ref):
    a = 1.0 / (1.0 + jnp.exp(-rw_ref[0, 0]))
    out_ref[...] = a * g_ref[...] + (1.0 - a) * z_ref[...]

    @pl.when(pl.program_id(0) == 0)
    def _():
        alpha_ref[...] = a.reshape(1, 1)


def _blend_call(rw2d, z_e, zq_pure):
    n, d = z_e.shape
    bn = 2048
    return pl.pallas_call(
        _blend_body,
        grid=(n // bn,),
        in_specs=[
            pl.BlockSpec((1, 1), lambda i: (0, 0)),
            pl.BlockSpec((bn, d), lambda i: (i, 0)),
            pl.BlockSpec((bn, d), lambda i: (i, 0)),
        ],
        out_specs=[
            pl.BlockSpec((bn, d), lambda i: (i, 0)),
            pl.BlockSpec((1, 1), lambda i: (0, 0)),
        ],
        out_shape=[
            jax.ShapeDtypeStruct((n, d), jnp.float32),
            jax.ShapeDtypeStruct((1, 1), jnp.float32),
        ],
        compiler_params=pltpu.CompilerParams(
            dimension_semantics=("arbitrary",)),
    )(rw2d, z_e, zq_pure)


def kernel(z_e, embeddings, logit_scale, residual_weight):
    n, d = z_e.shape
    scale2d = jnp.reshape(logit_scale, (1, 1)).astype(jnp.float32)
    rw2d = jnp.reshape(residual_weight, (1, 1)).astype(jnp.float32)

    idx_col, ppl = _stats_call(z_e, embeddings, scale2d)
    indices = jnp.reshape(idx_col, (n,))

    zq_pure = _gather_call(embeddings, jnp.reshape(indices, (-1, 128)), n, d)
    z_q, alpha2d = _blend_call(rw2d, z_e, zq_pure)

    perplexity = jnp.reshape(ppl, ())
    alpha = jnp.reshape(alpha2d, ())
    commitment_loss = jnp.zeros((), jnp.float32)
    return (z_q, indices, perplexity, alpha, commitment_loss)


# speedup vs baseline: 1.1723x; 1.1551x over previous
import functools

import jax
import jax.numpy as jnp
from jax import lax
from jax.experimental import pallas as pl
from jax.experimental.pallas import tpu as pltpu
from jax.experimental.pallas import tpu_sc as plsc

_BN = 256
_BK = 2048


def _stats_body(scale_ref, z_ref, emb_ref, idx_ref, ppl_ref,
                en_full, zn_s, ebuf, sacc, mval, midx, avg_acc, sem,
                *, n_total, k_total, bn, bk, nb_count, kb_count):
    nb = pl.program_id(0)
    kb = pl.program_id(1)

    @pl.when((nb == 0) & (kb == 0))
    def _init_once():
        pltpu.make_async_copy(emb_ref, en_full, sem).start()
        avg_acc[...] = jnp.zeros_like(avg_acc)
        pltpu.make_async_copy(emb_ref, en_full, sem).wait()

    @pl.when(nb == 0)
    def _norm_emb_block():
        e = en_full[pl.ds(kb * bk, bk), :]
        en_full[pl.ds(kb * bk, bk), :] = e / jnp.maximum(
            jnp.sqrt(jnp.sum(e * e, axis=1, keepdims=True)), 1e-12)

    scale = scale_ref[0, 0]

    @pl.when(kb == 0)
    def _init_row():
        z = z_ref[...]
        zn_s[...] = z / jnp.maximum(
            jnp.sqrt(jnp.sum(z * z, axis=1, keepdims=True)), 1e-12)
        sacc[...] = jnp.zeros_like(sacc)
        mval[...] = jnp.full_like(mval, -jnp.inf)
        midx[...] = jnp.zeros_like(midx)

    zn = zn_s[...]
    en = en_full[pl.ds(kb * bk, bk), :]
    logits = scale * lax.dot_general(
        zn, en, (((1,), (1,)), ((), ())),
        preferred_element_type=jnp.float32)
    eexp = jnp.exp(logits - jnp.abs(scale))
    ebuf[:, pl.ds(kb * bk, bk)] = eexp
    ones_col = jnp.ones((bk, 1), jnp.float32)
    sacc[...] += lax.dot_general(
        eexp, ones_col, (((1,), (0,)), ((), ())),
        preferred_element_type=jnp.float32)

    tmax = jnp.max(logits, axis=1, keepdims=True)
    colidx = lax.broadcasted_iota(jnp.int32, (bn, bk), 1)
    cand = jnp.min(jnp.where(logits == tmax, colidx, k_total),
                   axis=1, keepdims=True) + kb * bk
    better = tmax > mval[...]
    midx[...] = jnp.where(better, cand, midx[...])
    mval[...] = jnp.where(better, tmax, mval[...])

    @pl.when(kb == kb_count - 1)
    def _finish_row():
        recip_row = jnp.transpose(1.0 / sacc[...])
        for j in range(kb_count):
            chunk = ebuf[:, pl.ds(j * bk, bk)]
            avg_acc[:, pl.ds(j * bk, bk)] += lax.dot_general(
                recip_row, chunk, (((1,), (0,)), ((), ())),
                preferred_element_type=jnp.float32)
        idx_ref[...] = midx[...]

        @pl.when(nb == nb_count - 1)
        def _finish_all():
            avg = avg_acc[...] / n_total
            ent = jnp.sum(avg * jnp.log(avg + 1e-10))
            ppl_ref[...] = jnp.exp(-ent).reshape(1, 1)


def _stats_call(z_e, embeddings, scale2d):
    n, d = z_e.shape
    k = embeddings.shape[0]
    nb_count = n // _BN
    kb_count = k // _BK
    body = functools.partial(
        _stats_body, n_total=n, k_total=k, bn=_BN, bk=_BK,
        nb_count=nb_count, kb_count=kb_count)
    return pl.pallas_call(
        body,
        grid=(nb_count, kb_count),
        in_specs=[
            pl.BlockSpec((1, 1), lambda i, j: (0, 0)),
            pl.BlockSpec((_BN, d), lambda i, j: (i, 0)),
            pl.BlockSpec(memory_space=pl.ANY),
        ],
        out_specs=[
            pl.BlockSpec((_BN, 1), lambda i, j: (i, 0)),
            pl.BlockSpec((1, 1), lambda i, j: (0, 0)),
        ],
        out_shape=[
            jax.ShapeDtypeStruct((n, 1), jnp.int32),
            jax.ShapeDtypeStruct((1, 1), jnp.float32),
        ],
        scratch_shapes=[
            pltpu.VMEM((k, d), jnp.float32),
            pltpu.VMEM((_BN, d), jnp.float32),
            pltpu.VMEM((_BN, k), jnp.float32),
            pltpu.VMEM((_BN, 1), jnp.float32),
            pltpu.VMEM((_BN, 1), jnp.float32),
            pltpu.VMEM((_BN, 1), jnp.int32),
            pltpu.VMEM((1, k), jnp.float32),
            pltpu.SemaphoreType.DMA,
        ],
        compiler_params=pltpu.CompilerParams(
            dimension_semantics=("arbitrary", "arbitrary")),
    )(scale2d, z_e, embeddings)


def _gather_call(table, idx2d, n, d):
    info = plsc.get_sparse_core_info()
    nc, ns = info.num_cores, info.num_subcores
    nw = nc * ns
    b_per_w = n // nw
    chunks = b_per_w // 128
    mesh = plsc.VectorSubcoreMesh(core_axis_name="c", subcore_axis_name="s")

    @functools.partial(
        pl.kernel, mesh=mesh,
        out_type=jax.ShapeDtypeStruct((n, d), jnp.float32),
        compiler_params=pltpu.CompilerParams(use_tc_tiling_on_sc=False),
        scratch_types=[
            pltpu.VMEM((chunks, 128), jnp.int32),
            pltpu.VMEM((b_per_w, d), jnp.float32),
            pltpu.SemaphoreType.DMA,
        ],
    )
    def _gather_kernel(table_hbm, idx_hbm, out_hbm, idx_v, rows_v, sem):
        wid = lax.axis_index("s") * nc + lax.axis_index("c")
        pltpu.sync_copy(idx_hbm.at[pl.ds(wid * chunks, chunks)], idx_v)
        copies = [
            pltpu.async_copy(table_hbm.at[idx_v.at[j]],
                             rows_v.at[pl.ds(j * 128, 128)], sem)
            for j in range(chunks)
        ]
        for c in copies:
            c.wait()
        pltpu.sync_copy(rows_v, out_hbm.at[pl.ds(wid * b_per_w, b_per_w)])

    return _gather_kernel(table, idx2d)


def _blend_body(rw_ref, z_ref, g_ref, out_ref, alpha_ref):
    a = 1.0 / (1.0 + jnp.exp(-rw_ref[0, 0]))
    out_ref[...] = a * g_ref[...] + (1.0 - a) * z_ref[...]

    @pl.when(pl.program_id(0) == 0)
    def _():
        alpha_ref[...] = a.reshape(1, 1)


def _blend_call(rw2d, z_e, zq_pure):
    n, d = z_e.shape
    bn = 2048
    return pl.pallas_call(
        _blend_body,
        grid=(n // bn,),
        in_specs=[
            pl.BlockSpec((1, 1), lambda i: (0, 0)),
            pl.BlockSpec((bn, d), lambda i: (i, 0)),
            pl.BlockSpec((bn, d), lambda i: (i, 0)),
        ],
        out_specs=[
            pl.BlockSpec((bn, d), lambda i: (i, 0)),
            pl.BlockSpec((1, 1), lambda i: (0, 0)),
        ],
        out_shape=[
            jax.ShapeDtypeStruct((n, d), jnp.float32),
            jax.ShapeDtypeStruct((1, 1), jnp.float32),
        ],
        compiler_params=pltpu.CompilerParams(
            dimension_semantics=("arbitrary",)),
    )(rw2d, z_e, zq_pure)


def kernel(z_e, embeddings, logit_scale, residual_weight):
    n, d = z_e.shape
    scale2d = jnp.reshape(logit_scale, (1, 1)).astype(jnp.float32)
    rw2d = jnp.reshape(residual_weight, (1, 1)).astype(jnp.float32)

    idx_col, ppl = _stats_call(z_e, embeddings, scale2d)
    indices = jnp.reshape(idx_col, (n,))

    zq_pure = _gather_call(embeddings, jnp.reshape(indices, (-1, 128)), n, d)
    z_q, alpha2d = _blend_call(rw2d, z_e, zq_pure)

    perplexity = jnp.reshape(ppl, ())
    alpha = jnp.reshape(alpha2d, ())
    commitment_loss = jnp.zeros((), jnp.float32)
    return (z_q, indices, perplexity, alpha, commitment_loss)


# speedup vs baseline: 1.3005x; 1.1094x over previous
import functools

import jax
import jax.numpy as jnp
from jax import lax
from jax.experimental import pallas as pl
from jax.experimental.pallas import tpu as pltpu
from jax.experimental.pallas import tpu_sc as plsc

_BN = 256
_BK = 2048


def _stats_body(scale_ref, z_ref, emb_ref, idx_ref, ppl_ref,
                en_full, avg_acc, sem,
                *, n_total, k_total, bn, nb_count):
    nb = pl.program_id(0)
    scale = scale_ref[0, 0]

    @pl.when(nb == 0)
    def _init_once():
        pltpu.make_async_copy(emb_ref, en_full, sem).start()
        avg_acc[...] = jnp.zeros_like(avg_acc)
        pltpu.make_async_copy(emb_ref, en_full, sem).wait()
        e = en_full[...]
        en_full[...] = e / jnp.maximum(
            jnp.sqrt(jnp.sum(e * e, axis=1, keepdims=True)), 1e-12)

    z = z_ref[...]
    zn = z / jnp.maximum(
        jnp.sqrt(jnp.sum(z * z, axis=1, keepdims=True)), 1e-12)
    en = en_full[...]
    logits = scale * lax.dot_general(
        zn, en, (((1,), (1,)), ((), ())),
        preferred_element_type=jnp.float32)
    eexp = jnp.exp(logits - jnp.abs(scale))
    ones_col = jnp.ones((k_total, 1), jnp.float32)
    srow = lax.dot_general(
        eexp, ones_col, (((1,), (0,)), ((), ())),
        preferred_element_type=jnp.float32)

    tmax = jnp.max(logits, axis=1, keepdims=True)
    colidx = lax.broadcasted_iota(jnp.int32, (bn, k_total), 1)
    idx_ref[...] = jnp.min(jnp.where(logits == tmax, colidx, k_total),
                           axis=1, keepdims=True)

    recip_row = jnp.transpose(1.0 / srow)
    avg_acc[...] += lax.dot_general(
        recip_row, eexp, (((1,), (0,)), ((), ())),
        preferred_element_type=jnp.float32)

    @pl.when(nb == nb_count - 1)
    def _finish_all():
        avg = avg_acc[...] / n_total
        ent = jnp.sum(avg * jnp.log(avg + 1e-10))
        ppl_ref[...] = jnp.exp(-ent).reshape(1, 1)


def _stats_call(z_e, embeddings, scale2d):
    n, d = z_e.shape
    k = embeddings.shape[0]
    nb_count = n // _BN
    body = functools.partial(
        _stats_body, n_total=n, k_total=k, bn=_BN, nb_count=nb_count)
    return pl.pallas_call(
        body,
        grid=(nb_count,),
        in_specs=[
            pl.BlockSpec((1, 1), lambda i: (0, 0)),
            pl.BlockSpec((_BN, d), lambda i: (i, 0)),
            pl.BlockSpec(memory_space=pl.ANY),
        ],
        out_specs=[
            pl.BlockSpec((_BN, 1), lambda i: (i, 0)),
            pl.BlockSpec((1, 1), lambda i: (0, 0)),
        ],
        out_shape=[
            jax.ShapeDtypeStruct((n, 1), jnp.int32),
            jax.ShapeDtypeStruct((1, 1), jnp.float32),
        ],
        scratch_shapes=[
            pltpu.VMEM((k, d), jnp.float32),
            pltpu.VMEM((1, k), jnp.float32),
            pltpu.SemaphoreType.DMA,
        ],
        compiler_params=pltpu.CompilerParams(
            dimension_semantics=("arbitrary",)),
    )(scale2d, z_e, embeddings)


def _gather_call(table, idx2d, n, d):
    info = plsc.get_sparse_core_info()
    nc, ns = info.num_cores, info.num_subcores
    nw = nc * ns
    b_per_w = n // nw
    chunks = b_per_w // 128
    mesh = plsc.VectorSubcoreMesh(core_axis_name="c", subcore_axis_name="s")

    @functools.partial(
        pl.kernel, mesh=mesh,
        out_type=jax.ShapeDtypeStruct((n, d), jnp.float32),
        compiler_params=pltpu.CompilerParams(use_tc_tiling_on_sc=False),
        scratch_types=[
            pltpu.VMEM((chunks, 128), jnp.int32),
            pltpu.VMEM((b_per_w, d), jnp.float32),
            pltpu.SemaphoreType.DMA,
        ],
    )
    def _gather_kernel(table_hbm, idx_hbm, out_hbm, idx_v, rows_v, sem):
        wid = lax.axis_index("s") * nc + lax.axis_index("c")
        pltpu.sync_copy(idx_hbm.at[pl.ds(wid * chunks, chunks)], idx_v)
        copies = [
            pltpu.async_copy(table_hbm.at[idx_v.at[j]],
                             rows_v.at[pl.ds(j * 128, 128)], sem)
            for j in range(chunks)
        ]
        for c in copies:
            c.wait()
        pltpu.sync_copy(rows_v, out_hbm.at[pl.ds(wid * b_per_w, b_per_w)])

    return _gather_kernel(table, idx2d)


def _blend_body(rw_ref, z_ref, g_ref, out_ref, alpha_ref):
    a = 1.0 / (1.0 + jnp.exp(-rw_ref[0, 0]))
    out_ref[...] = a * g_ref[...] + (1.0 - a) * z_ref[...]

    @pl.when(pl.program_id(0) == 0)
    def _():
        alpha_ref[...] = a.reshape(1, 1)


def _blend_call(rw2d, z_e, zq_pure):
    n, d = z_e.shape
    bn = 2048
    return pl.pallas_call(
        _blend_body,
        grid=(n // bn,),
        in_specs=[
            pl.BlockSpec((1, 1), lambda i: (0, 0)),
            pl.BlockSpec((bn, d), lambda i: (i, 0)),
            pl.BlockSpec((bn, d), lambda i: (i, 0)),
        ],
        out_specs=[
            pl.BlockSpec((bn, d), lambda i: (i, 0)),
            pl.BlockSpec((1, 1), lambda i: (0, 0)),
        ],
        out_shape=[
            jax.ShapeDtypeStruct((n, d), jnp.float32),
            jax.ShapeDtypeStruct((1, 1), jnp.float32),
        ],
        compiler_params=pltpu.CompilerParams(
            dimension_semantics=("arbitrary",)),
    )(rw2d, z_e, zq_pure)


def kernel(z_e, embeddings, logit_scale, residual_weight):
    n, d = z_e.shape
    scale2d = jnp.reshape(logit_scale, (1, 1)).astype(jnp.float32)
    rw2d = jnp.reshape(residual_weight, (1, 1)).astype(jnp.float32)

    idx_col, ppl = _stats_call(z_e, embeddings, scale2d)
    indices = jnp.reshape(idx_col, (n,))

    zq_pure = _gather_call(embeddings, jnp.reshape(indices, (-1, 128)), n, d)
    z_q, alpha2d = _blend_call(rw2d, z_e, zq_pure)

    perplexity = jnp.reshape(ppl, ())
    alpha = jnp.reshape(alpha2d, ())
    commitment_loss = jnp.zeros((), jnp.float32)
    return (z_q, indices, perplexity, alpha, commitment_loss)


# speedup vs baseline: 1.3382x; 1.0291x over previous
import functools

import jax
import jax.numpy as jnp
from jax import lax
from jax.experimental import pallas as pl
from jax.experimental.pallas import tpu as pltpu
from jax.experimental.pallas import tpu_sc as plsc

_BN = 512
_BK = 2048


def _stats_body(scale_ref, z_ref, emb_ref, idx_ref, ppl_ref,
                en_full, avg_acc, sem,
                *, n_total, k_total, bn, nb_count):
    nb = pl.program_id(0)
    scale = scale_ref[0, 0]

    @pl.when(nb == 0)
    def _init_once():
        pltpu.make_async_copy(emb_ref, en_full, sem).start()
        avg_acc[...] = jnp.zeros_like(avg_acc)
        pltpu.make_async_copy(emb_ref, en_full, sem).wait()
        e = en_full[...]
        en_full[...] = e / jnp.maximum(
            jnp.sqrt(jnp.sum(e * e, axis=1, keepdims=True)), 1e-12)

    z = z_ref[...]
    zn = z / jnp.maximum(
        jnp.sqrt(jnp.sum(z * z, axis=1, keepdims=True)), 1e-12)
    en = en_full[...]
    logits = scale * lax.dot_general(
        zn, en, (((1,), (1,)), ((), ())),
        preferred_element_type=jnp.float32)
    eexp = jnp.exp(logits - jnp.abs(scale))
    ones_col = jnp.ones((k_total, 1), jnp.float32)
    srow = lax.dot_general(
        eexp, ones_col, (((1,), (0,)), ((), ())),
        preferred_element_type=jnp.float32)

    tmax = jnp.max(logits, axis=1, keepdims=True)
    colidx = lax.broadcasted_iota(jnp.int32, (bn, k_total), 1)
    idx_ref[...] = jnp.min(jnp.where(logits == tmax, colidx, k_total),
                           axis=1, keepdims=True)

    recip_row = jnp.transpose(1.0 / srow)
    avg_acc[...] += lax.dot_general(
        recip_row, eexp, (((1,), (0,)), ((), ())),
        preferred_element_type=jnp.float32)

    @pl.when(nb == nb_count - 1)
    def _finish_all():
        avg = avg_acc[...] / n_total
        ent = jnp.sum(avg * jnp.log(avg + 1e-10))
        ppl_ref[...] = jnp.exp(-ent).reshape(1, 1)


def _stats_call(z_e, embeddings, scale2d):
    n, d = z_e.shape
    k = embeddings.shape[0]
    nb_count = n // _BN
    body = functools.partial(
        _stats_body, n_total=n, k_total=k, bn=_BN, nb_count=nb_count)
    return pl.pallas_call(
        body,
        grid=(nb_count,),
        in_specs=[
            pl.BlockSpec((1, 1), lambda i: (0, 0)),
            pl.BlockSpec((_BN, d), lambda i: (i, 0)),
            pl.BlockSpec(memory_space=pl.ANY),
        ],
        out_specs=[
            pl.BlockSpec((_BN, 1), lambda i: (i, 0)),
            pl.BlockSpec((1, 1), lambda i: (0, 0)),
        ],
        out_shape=[
            jax.ShapeDtypeStruct((n, 1), jnp.int32),
            jax.ShapeDtypeStruct((1, 1), jnp.float32),
        ],
        scratch_shapes=[
            pltpu.VMEM((k, d), jnp.float32),
            pltpu.VMEM((1, k), jnp.float32),
            pltpu.SemaphoreType.DMA,
        ],
        compiler_params=pltpu.CompilerParams(
            dimension_semantics=("arbitrary",)),
    )(scale2d, z_e, embeddings)


def _gather_call(table, idx2d, n, d):
    info = plsc.get_sparse_core_info()
    nc, ns = info.num_cores, info.num_subcores
    nw = nc * ns
    b_per_w = n // nw
    chunks = b_per_w // 128
    mesh = plsc.VectorSubcoreMesh(core_axis_name="c", subcore_axis_name="s")

    @functools.partial(
        pl.kernel, mesh=mesh,
        out_type=jax.ShapeDtypeStruct((n, d), jnp.float32),
        compiler_params=pltpu.CompilerParams(use_tc_tiling_on_sc=False),
        scratch_types=[
            pltpu.VMEM((chunks, 128), jnp.int32),
            pltpu.VMEM((b_per_w, d), jnp.float32),
            pltpu.SemaphoreType.DMA,
        ],
    )
    def _gather_kernel(table_hbm, idx_hbm, out_hbm, idx_v, rows_v, sem):
        wid = lax.axis_index("s") * nc + lax.axis_index("c")
        pltpu.sync_copy(idx_hbm.at[pl.ds(wid * chunks, chunks)], idx_v)
        copies = [
            pltpu.async_copy(table_hbm.at[idx_v.at[j]],
                             rows_v.at[pl.ds(j * 128, 128)], sem)
            for j in range(chunks)
        ]
        for c in copies:
            c.wait()
        pltpu.sync_copy(rows_v, out_hbm.at[pl.ds(wid * b_per_w, b_per_w)])

    return _gather_kernel(table, idx2d)


def _blend_body(rw_ref, z_ref, g_ref, out_ref, alpha_ref):
    a = 1.0 / (1.0 + jnp.exp(-rw_ref[0, 0]))
    out_ref[...] = a * g_ref[...] + (1.0 - a) * z_ref[...]

    @pl.when(pl.program_id(0) == 0)
    def _():
        alpha_ref[...] = a.reshape(1, 1)


def _blend_call(rw2d, z_e, zq_pure):
    n, d = z_e.shape
    bn = 2048
    return pl.pallas_call(
        _blend_body,
        grid=(n // bn,),
        in_specs=[
            pl.BlockSpec((1, 1), lambda i: (0, 0)),
            pl.BlockSpec((bn, d), lambda i: (i, 0)),
            pl.BlockSpec((bn, d), lambda i: (i, 0)),
        ],
        out_specs=[
            pl.BlockSpec((bn, d), lambda i: (i, 0)),
            pl.BlockSpec((1, 1), lambda i: (0, 0)),
        ],
        out_shape=[
            jax.ShapeDtypeStruct((n, d), jnp.float32),
            jax.ShapeDtypeStruct((1, 1), jnp.float32),
        ],
        compiler_params=pltpu.CompilerParams(
            dimension_semantics=("arbitrary",)),
    )(rw2d, z_e, zq_pure)


def kernel(z_e, embeddings, logit_scale, residual_weight):
    n, d = z_e.shape
    scale2d = jnp.reshape(logit_scale, (1, 1)).astype(jnp.float32)
    rw2d = jnp.reshape(residual_weight, (1, 1)).astype(jnp.float32)

    idx_col, ppl = _stats_call(z_e, embeddings, scale2d)
    indices = jnp.reshape(idx_col, (n,))

    zq_pure = _gather_call(embeddings, jnp.reshape(indices, (-1, 128)), n, d)
    z_q, alpha2d = _blend_call(rw2d, z_e, zq_pure)

    perplexity = jnp.reshape(ppl, ())
    alpha = jnp.reshape(alpha2d, ())
    commitment_loss = jnp.zeros((), jnp.float32)
    return (z_q, indices, perplexity, alpha, commitment_loss)
